# Initial kernel scaffold; baseline (speedup 1.0000x reference)
import jax
import jax.numpy as jnp
from jax.experimental import pallas as pl


def kernel(input_seq, seq_lengths, supports1, supports2, W1g0, b1g0, W1c0, b1c0, W1g1, b1g1, W1c1, b1c1, W2g0, b2g0, W2c0, b2c0, W2g1, b2g1, W2c1, b2c1, fcW, fcb):
    raise NotImplementedError("write your pallas kernel here")


# speedup vs baseline: 7.0881x; 7.0881x over previous
import jax
import jax.numpy as jnp
from jax.experimental import pallas as pl
from jax.experimental.pallas import tpu as pltpu

_N = 207
_D_IN = 2
_HID = 64
_CLS = 4
_GRID = 4


def _mm(a, b):
    return jax.lax.dot_general(a, b, (((1,), (0,)), ((), ())),
                               preferred_element_type=jnp.float32)


def _sdot(S, x):
    return jax.lax.dot_general(S, x, (((1,), (0,)), ((), ())),
                               preferred_element_type=jnp.float32)


def _cdot(x, w):
    return jax.lax.dot_general(x, w, (((2,), (0,)), ((), ())),
                               preferred_element_type=jnp.float32)


def _body(x_ref, oh_ref, s1_ref, s2_ref,
          w1g0x_ref, w1g0h_ref, b1g0_ref, w1c0x_ref, w1c0h_ref, b1c0_ref,
          w1g1x_ref, w1g1h_ref, b1g1_ref, w1c1x_ref, w1c1h_ref, b1c1_ref,
          w2g0x_ref, w2g0h_ref, b2g0_ref, w2c0x_ref, w2c0h_ref, b2c0_ref,
          w2g1x_ref, w2g1h_ref, b2g1_ref, w2c1x_ref, w2c1h_ref, b2c1_ref,
          fcw_ref, fcb_ref, o_ref,
          ch1_ref, h01_ref, h11_ref, l1_ref,
          ch2_ref, h02_ref, h12_ref, l2_ref):
    T = x_ref.shape[0]
    B = oh_ref.shape[2]

    def chebcat(S, x0):
        x1 = _sdot(S, x0)
        x2 = 2.0 * _sdot(S, x1) - x0
        return jnp.concatenate([x0, x1, x2], axis=-1)

    for ref in (ch1_ref, ch2_ref, h01_ref, h02_ref,
                h11_ref, h12_ref, l1_ref, l2_ref):
        ref[...] = jnp.zeros_like(ref)

    encs = (
        (s1_ref, ch1_ref, h01_ref, h11_ref, l1_ref,
         w1g0x_ref, w1g0h_ref, b1g0_ref, w1c0x_ref, w1c0h_ref, b1c0_ref,
         w1g1x_ref, w1g1h_ref, b1g1_ref, w1c1x_ref, w1c1h_ref, b1c1_ref),
        (s2_ref, ch2_ref, h02_ref, h12_ref, l2_ref,
         w2g0x_ref, w2g0h_ref, b2g0_ref, w2c0x_ref, w2c0h_ref, b2c0_ref,
         w2g1x_ref, w2g1h_ref, b2g1_ref, w2c1x_ref, w2c1h_ref, b2c1_ref),
    )

    def step(t, carry):
        xw_t = x_ref[t, 0]
        x_t = xw_t.reshape(_N, B, _D_IN)
        m = oh_ref[t]
        for (s_ref, ch_ref, h0_ref, h1_ref, l_ref,
             wg0x, wg0h, bg0, wc0x, wc0h, bc0,
             wg1x, wg1h, bg1, wc1x, wc1h, bc1) in encs:
            S = s_ref[...]
            xcat0 = chebcat(S, x_t)
            hcat0 = ch_ref[...]
            h0 = h0_ref[...]
            g = jax.nn.sigmoid(_cdot(xcat0, wg0x[...]) +
                               _cdot(hcat0, wg0h[...]) + bg0[...])
            r = g[:, :, :_HID]
            u = g[:, :, _HID:]
            ccat = chebcat(S, r * h0)
            c = jnp.tanh(_cdot(xcat0, wc0x[...]) +
                         _cdot(ccat, wc0h[...]) + bc0[...])
            h0n = u * h0 + (1.0 - u) * c
            xcat1 = chebcat(S, h0n)
            ch_ref[...] = xcat1
            h0_ref[...] = h0n
            h1 = h1_ref[...]
            hcat1 = chebcat(S, h1)
            g1 = jax.nn.sigmoid(_cdot(xcat1, wg1x[...]) +
                                _cdot(hcat1, wg1h[...]) + bg1[...])
            r1 = g1[:, :, :_HID]
            u1 = g1[:, :, _HID:]
            ccat1 = chebcat(S, r1 * h1)
            c1 = jnp.tanh(_cdot(xcat1, wc1x[...]) +
                          _cdot(ccat1, wc1h[...]) + bc1[...])
            h1n = u1 * h1 + (1.0 - u1) * c1
            h1_ref[...] = h1n
            l_ref[...] = m * h1n + (1.0 - m) * l_ref[...]
        return carry

    jax.lax.fori_loop(0, T, step, 0, unroll=False)

    ua = jnp.maximum(
        jnp.concatenate([l1_ref[...], l2_ref[...]], axis=0), 0.0)
    pq = _cdot(ua, fcw_ref[...])
    pq = pq.reshape(_N, 2, B, 2 * _CLS)
    logits = pq[:, 0, :, :_CLS] + pq[:, 1, :, _CLS:]
    o_ref[...] = jnp.max(logits, axis=0) + fcb_ref[...]


def _prep_w(W, cin):
    out = W.shape[1]
    w3 = W.reshape(cin, 3, out)
    wx = w3[:cin - _HID].transpose(1, 0, 2).reshape(3 * (cin - _HID), out)
    wh = w3[cin - _HID:].transpose(1, 0, 2).reshape(3 * _HID, out)
    return wx, wh


def kernel(input_seq, seq_lengths, supports1, supports2,
           W1g0, b1g0, W1c0, b1c0, W1g1, b1g1, W1c1, b1c1,
           W2g0, b2g0, W2c0, b2c0, W2g1, b2g1, W2c1, b2c1,
           fcW, fcb):
    B, T = input_seq.shape[0], input_seq.shape[1]
    BS = B // _GRID
    xseq = (input_seq.transpose(1, 2, 0, 3)
            .reshape(T, _N, _GRID, BS * _D_IN)
            .transpose(0, 2, 1, 3))
    idx = jnp.clip(seq_lengths - 1, 0, T - 1).astype(jnp.int32)
    oh = (idx[None, :] == jnp.arange(T, dtype=jnp.int32)[:, None])
    oh = oh.astype(jnp.float32).reshape(T, 1, B, 1)

    c0 = _D_IN + _HID
    c1 = 2 * _HID
    f32 = jnp.float32

    def fixed(shape):
        nd = len(shape)
        return pl.BlockSpec(shape, lambda i, _n=nd: (0,) * _n)

    w1g0x, w1g0h = _prep_w(W1g0, c0)
    w1c0x, w1c0h = _prep_w(W1c0, c0)
    w1g1x, w1g1h = _prep_w(W1g1, c1)
    w1c1x, w1c1h = _prep_w(W1c1, c1)
    w2g0x, w2g0h = _prep_w(W2g0, c0)
    w2c0x, w2c0h = _prep_w(W2c0, c0)
    w2g1x, w2g1h = _prep_w(W2g1, c1)
    w2c1x, w2c1h = _prep_w(W2c1, c1)

    args = (
        xseq, oh, supports1, supports2,
        w1g0x, w1g0h, b1g0.reshape(1, 1, -1),
        w1c0x, w1c0h, b1c0.reshape(1, 1, -1),
        w1g1x, w1g1h, b1g1.reshape(1, 1, -1),
        w1c1x, w1c1h, b1c1.reshape(1, 1, -1),
        w2g0x, w2g0h, b2g0.reshape(1, 1, -1),
        w2c0x, w2c0h, b2c0.reshape(1, 1, -1),
        w2g1x, w2g1h, b2g1.reshape(1, 1, -1),
        w2c1x, w2c1h, b2c1.reshape(1, 1, -1),
        jnp.concatenate([fcW[:_HID], fcW[_HID:]], axis=1),
        fcb.reshape(1, -1),
    )
    in_specs = [
        pl.BlockSpec((T, 1, _N, BS * _D_IN), lambda i: (0, i, 0, 0)),
        pl.BlockSpec((T, 1, BS, 1), lambda i: (0, 0, i, 0)),
    ] + [fixed(a.shape) for a in args[2:]]

    out = pl.pallas_call(
        _body,
        grid=(_GRID,),
        out_shape=jax.ShapeDtypeStruct((B, _CLS), f32),
        in_specs=in_specs,
        out_specs=pl.BlockSpec((BS, _CLS), lambda i: (i, 0)),
        scratch_shapes=[
            pltpu.VMEM((_N, BS, 3 * _HID), f32),
            pltpu.VMEM((_N, BS, _HID), f32),
            pltpu.VMEM((_N, BS, _HID), f32),
            pltpu.VMEM((_N, BS, _HID), f32),
            pltpu.VMEM((_N, BS, 3 * _HID), f32),
            pltpu.VMEM((_N, BS, _HID), f32),
            pltpu.VMEM((_N, BS, _HID), f32),
            pltpu.VMEM((_N, BS, _HID), f32),
        ],
        compiler_params=pltpu.CompilerParams(
            vmem_limit_bytes=100 * 1024 * 1024),
    )(*args)
    return out


# speedup vs baseline: 7.6873x; 1.0845x over previous
import jax
import jax.numpy as jnp
from jax.experimental import pallas as pl
from jax.experimental.pallas import tpu as pltpu

_N = 207
_D_IN = 2
_HID = 64
_CLS = 4
_GRID = 4


def _mm(a, b):
    return jax.lax.dot_general(a, b, (((1,), (0,)), ((), ())),
                               preferred_element_type=jnp.float32)


def _sdot(S, x):
    return jax.lax.dot_general(S, x.astype(jnp.bfloat16),
                               (((1,), (0,)), ((), ())),
                               preferred_element_type=jnp.float32)


def _cdot(x, w):
    return jax.lax.dot_general(x.astype(jnp.bfloat16), w,
                               (((2,), (0,)), ((), ())),
                               preferred_element_type=jnp.float32)


def _body(x_ref, oh_ref, s1_ref, s2_ref,
          w1g0x_ref, w1g0h_ref, b1g0_ref, w1c0x_ref, w1c0h_ref, b1c0_ref,
          w1g1x_ref, w1g1h_ref, b1g1_ref, w1c1x_ref, w1c1h_ref, b1c1_ref,
          w2g0x_ref, w2g0h_ref, b2g0_ref, w2c0x_ref, w2c0h_ref, b2c0_ref,
          w2g1x_ref, w2g1h_ref, b2g1_ref, w2c1x_ref, w2c1h_ref, b2c1_ref,
          fcw_ref, fcb_ref, o_ref,
          ch1_ref, h01_ref, h11_ref, l1_ref,
          ch2_ref, h02_ref, h12_ref, l2_ref):
    T = x_ref.shape[0]
    B = oh_ref.shape[2]

    def chebcat(S, x0):
        x1 = _sdot(S, x0)
        x2 = 2.0 * _sdot(S, x1) - x0
        return jnp.concatenate([x0, x1, x2], axis=-1)

    for ref in (ch1_ref, ch2_ref, h01_ref, h02_ref,
                h11_ref, h12_ref, l1_ref, l2_ref):
        ref[...] = jnp.zeros_like(ref)

    encs = (
        (s1_ref, ch1_ref, h01_ref, h11_ref, l1_ref,
         w1g0x_ref, w1g0h_ref, b1g0_ref, w1c0x_ref, w1c0h_ref, b1c0_ref,
         w1g1x_ref, w1g1h_ref, b1g1_ref, w1c1x_ref, w1c1h_ref, b1c1_ref),
        (s2_ref, ch2_ref, h02_ref, h12_ref, l2_ref,
         w2g0x_ref, w2g0h_ref, b2g0_ref, w2c0x_ref, w2c0h_ref, b2c0_ref,
         w2g1x_ref, w2g1h_ref, b2g1_ref, w2c1x_ref, w2c1h_ref, b2c1_ref),
    )

    def step(t, carry):
        xw_t = x_ref[t, 0]
        x_t = xw_t.reshape(_N, B, _D_IN)
        m = oh_ref[t]
        for (s_ref, ch_ref, h0_ref, h1_ref, l_ref,
             wg0x, wg0h, bg0, wc0x, wc0h, bc0,
             wg1x, wg1h, bg1, wc1x, wc1h, bc1) in encs:
            S = s_ref[...]
            xcat0 = chebcat(S, x_t)
            hcat0 = ch_ref[...]
            h0 = h0_ref[...]
            g = jax.nn.sigmoid(_cdot(xcat0, wg0x[...]) +
                               _cdot(hcat0, wg0h[...]) + bg0[...])
            r = g[:, :, :_HID]
            u = g[:, :, _HID:]
            ccat = chebcat(S, r * h0)
            c = jnp.tanh(_cdot(xcat0, wc0x[...]) +
                         _cdot(ccat, wc0h[...]) + bc0[...])
            h0n = u * h0 + (1.0 - u) * c
            xcat1 = chebcat(S, h0n)
            ch_ref[...] = xcat1
            h0_ref[...] = h0n
            h1 = h1_ref[...]
            hcat1 = chebcat(S, h1)
            g1 = jax.nn.sigmoid(_cdot(xcat1, wg1x[...]) +
                                _cdot(hcat1, wg1h[...]) + bg1[...])
            r1 = g1[:, :, :_HID]
            u1 = g1[:, :, _HID:]
            ccat1 = chebcat(S, r1 * h1)
            c1 = jnp.tanh(_cdot(xcat1, wc1x[...]) +
                          _cdot(ccat1, wc1h[...]) + bc1[...])
            h1n = u1 * h1 + (1.0 - u1) * c1
            h1_ref[...] = h1n
            l_ref[...] = m * h1n + (1.0 - m) * l_ref[...]
        return carry

    jax.lax.fori_loop(0, T, step, 0, unroll=False)

    ua = jnp.maximum(
        jnp.concatenate([l1_ref[...], l2_ref[...]], axis=0), 0.0)
    pq = _cdot(ua, fcw_ref[...])
    pq = pq.reshape(_N, 2, B, 2 * _CLS)
    logits = pq[:, 0, :, :_CLS] + pq[:, 1, :, _CLS:]
    o_ref[...] = jnp.max(logits, axis=0) + fcb_ref[...]


def _prep_w(W, cin):
    out = W.shape[1]
    w3 = W.reshape(cin, 3, out)
    wx = w3[:cin - _HID].transpose(1, 0, 2).reshape(3 * (cin - _HID), out)
    wh = w3[cin - _HID:].transpose(1, 0, 2).reshape(3 * _HID, out)
    return wx, wh


def kernel(input_seq, seq_lengths, supports1, supports2,
           W1g0, b1g0, W1c0, b1c0, W1g1, b1g1, W1c1, b1c1,
           W2g0, b2g0, W2c0, b2c0, W2g1, b2g1, W2c1, b2c1,
           fcW, fcb):
    B, T = input_seq.shape[0], input_seq.shape[1]
    BS = B // _GRID
    xseq = (input_seq.transpose(1, 2, 0, 3)
            .reshape(T, _N, _GRID, BS * _D_IN)
            .transpose(0, 2, 1, 3))
    idx = jnp.clip(seq_lengths - 1, 0, T - 1).astype(jnp.int32)
    oh = (idx[None, :] == jnp.arange(T, dtype=jnp.int32)[:, None])
    oh = oh.astype(jnp.float32).reshape(T, 1, B, 1)

    c0 = _D_IN + _HID
    c1 = 2 * _HID
    f32 = jnp.float32

    def fixed(shape):
        nd = len(shape)
        return pl.BlockSpec(shape, lambda i, _n=nd: (0,) * _n)

    bf16 = jnp.bfloat16
    w1g0x, w1g0h = _prep_w(W1g0.astype(bf16), c0)
    w1c0x, w1c0h = _prep_w(W1c0.astype(bf16), c0)
    w1g1x, w1g1h = _prep_w(W1g1.astype(bf16), c1)
    w1c1x, w1c1h = _prep_w(W1c1.astype(bf16), c1)
    w2g0x, w2g0h = _prep_w(W2g0.astype(bf16), c0)
    w2c0x, w2c0h = _prep_w(W2c0.astype(bf16), c0)
    w2g1x, w2g1h = _prep_w(W2g1.astype(bf16), c1)
    w2c1x, w2c1h = _prep_w(W2c1.astype(bf16), c1)

    args = (
        xseq, oh, supports1.astype(bf16), supports2.astype(bf16),
        w1g0x, w1g0h, b1g0.reshape(1, 1, -1),
        w1c0x, w1c0h, b1c0.reshape(1, 1, -1),
        w1g1x, w1g1h, b1g1.reshape(1, 1, -1),
        w1c1x, w1c1h, b1c1.reshape(1, 1, -1),
        w2g0x, w2g0h, b2g0.reshape(1, 1, -1),
        w2c0x, w2c0h, b2c0.reshape(1, 1, -1),
        w2g1x, w2g1h, b2g1.reshape(1, 1, -1),
        w2c1x, w2c1h, b2c1.reshape(1, 1, -1),
        jnp.concatenate([fcW[:_HID], fcW[_HID:]],
                        axis=1).astype(bf16),
        fcb.reshape(1, -1),
    )
    in_specs = [
        pl.BlockSpec((T, 1, _N, BS * _D_IN), lambda i: (0, i, 0, 0)),
        pl.BlockSpec((T, 1, BS, 1), lambda i: (0, 0, i, 0)),
    ] + [fixed(a.shape) for a in args[2:]]

    out = pl.pallas_call(
        _body,
        grid=(_GRID,),
        out_shape=jax.ShapeDtypeStruct((B, _CLS), f32),
        in_specs=in_specs,
        out_specs=pl.BlockSpec((BS, _CLS), lambda i: (i, 0)),
        scratch_shapes=[
            pltpu.VMEM((_N, BS, 3 * _HID), f32),
            pltpu.VMEM((_N, BS, _HID), f32),
            pltpu.VMEM((_N, BS, _HID), f32),
            pltpu.VMEM((_N, BS, _HID), f32),
            pltpu.VMEM((_N, BS, 3 * _HID), f32),
            pltpu.VMEM((_N, BS, _HID), f32),
            pltpu.VMEM((_N, BS, _HID), f32),
            pltpu.VMEM((_N, BS, _HID), f32),
        ],
        compiler_params=pltpu.CompilerParams(
            vmem_limit_bytes=100 * 1024 * 1024),
    )(*args)
    return out


# speedup vs baseline: 7.9892x; 1.0393x over previous
import jax
import jax.numpy as jnp
from jax.experimental import pallas as pl
from jax.experimental.pallas import tpu as pltpu

_N = 207
_D_IN = 2
_HID = 64
_CLS = 4
_GRID = 2


def _mm(a, b):
    return jax.lax.dot_general(a, b, (((1,), (0,)), ((), ())),
                               preferred_element_type=jnp.float32)


def _sdot(S, x):
    return jax.lax.dot_general(S, x.astype(jnp.bfloat16),
                               (((1,), (0,)), ((), ())),
                               preferred_element_type=jnp.float32)


def _cdot(x, w):
    return jax.lax.dot_general(x.astype(jnp.bfloat16), w,
                               (((2,), (0,)), ((), ())),
                               preferred_element_type=jnp.float32)


def _body(x_ref, oh_ref, s1_ref, s2_ref,
          w1g0x_ref, w1g0h_ref, b1g0_ref, w1c0x_ref, w1c0h_ref, b1c0_ref,
          w1g1x_ref, w1g1h_ref, b1g1_ref, w1c1x_ref, w1c1h_ref, b1c1_ref,
          w2g0x_ref, w2g0h_ref, b2g0_ref, w2c0x_ref, w2c0h_ref, b2c0_ref,
          w2g1x_ref, w2g1h_ref, b2g1_ref, w2c1x_ref, w2c1h_ref, b2c1_ref,
          fcw_ref, fcb_ref, o_ref,
          ch1_ref, h01_ref, h11_ref, l1_ref,
          ch2_ref, h02_ref, h12_ref, l2_ref):
    T = x_ref.shape[0]
    B = oh_ref.shape[2]

    def chebcat(S, x0):
        x1 = _sdot(S, x0)
        x2 = 2.0 * _sdot(S, x1) - x0
        return jnp.concatenate([x0, x1, x2], axis=-1)

    for ref in (ch1_ref, ch2_ref, h01_ref, h02_ref,
                h11_ref, h12_ref, l1_ref, l2_ref):
        ref[...] = jnp.zeros_like(ref)

    encs = (
        (s1_ref, ch1_ref, h01_ref, h11_ref, l1_ref,
         w1g0x_ref, w1g0h_ref, b1g0_ref, w1c0x_ref, w1c0h_ref, b1c0_ref,
         w1g1x_ref, w1g1h_ref, b1g1_ref, w1c1x_ref, w1c1h_ref, b1c1_ref),
        (s2_ref, ch2_ref, h02_ref, h12_ref, l2_ref,
         w2g0x_ref, w2g0h_ref, b2g0_ref, w2c0x_ref, w2c0h_ref, b2c0_ref,
         w2g1x_ref, w2g1h_ref, b2g1_ref, w2c1x_ref, w2c1h_ref, b2c1_ref),
    )

    def step(t, carry):
        xw_t = x_ref[t, 0]
        x_t = xw_t.reshape(_N, B, _D_IN)
        m = oh_ref[t]
        for (s_ref, ch_ref, h0_ref, h1_ref, l_ref,
             wg0x, wg0h, bg0, wc0x, wc0h, bc0,
             wg1x, wg1h, bg1, wc1x, wc1h, bc1) in encs:
            S = s_ref[...]
            xcat0 = chebcat(S, x_t)
            hcat0 = ch_ref[...]
            h0 = h0_ref[...]
            g = jax.nn.sigmoid(_cdot(xcat0, wg0x[...]) +
                               _cdot(hcat0, wg0h[...]) + bg0[...])
            r = g[:, :, :_HID]
            u = g[:, :, _HID:]
            ccat = chebcat(S, r * h0)
            c = jnp.tanh(_cdot(xcat0, wc0x[...]) +
                         _cdot(ccat, wc0h[...]) + bc0[...])
            h0n = u * h0 + (1.0 - u) * c
            xcat1 = chebcat(S, h0n)
            ch_ref[...] = xcat1
            h0_ref[...] = h0n
            h1 = h1_ref[...]
            hcat1 = chebcat(S, h1)
            g1 = jax.nn.sigmoid(_cdot(xcat1, wg1x[...]) +
                                _cdot(hcat1, wg1h[...]) + bg1[...])
            r1 = g1[:, :, :_HID]
            u1 = g1[:, :, _HID:]
            ccat1 = chebcat(S, r1 * h1)
            c1 = jnp.tanh(_cdot(xcat1, wc1x[...]) +
                          _cdot(ccat1, wc1h[...]) + bc1[...])
            h1n = u1 * h1 + (1.0 - u1) * c1
            h1_ref[...] = h1n
            l_ref[...] = m * h1n + (1.0 - m) * l_ref[...]
        return carry

    jax.lax.fori_loop(0, T, step, 0, unroll=False)

    ua = jnp.maximum(
        jnp.concatenate([l1_ref[...], l2_ref[...]], axis=0), 0.0)
    pq = _cdot(ua, fcw_ref[...])
    pq = pq.reshape(_N, 2, B, 2 * _CLS)
    logits = pq[:, 0, :, :_CLS] + pq[:, 1, :, _CLS:]
    o_ref[...] = jnp.max(logits, axis=0) + fcb_ref[...]


def _prep_w(W, cin):
    out = W.shape[1]
    w3 = W.reshape(cin, 3, out)
    wx = w3[:cin - _HID].transpose(1, 0, 2).reshape(3 * (cin - _HID), out)
    wh = w3[cin - _HID:].transpose(1, 0, 2).reshape(3 * _HID, out)
    return wx, wh


def kernel(input_seq, seq_lengths, supports1, supports2,
           W1g0, b1g0, W1c0, b1c0, W1g1, b1g1, W1c1, b1c1,
           W2g0, b2g0, W2c0, b2c0, W2g1, b2g1, W2c1, b2c1,
           fcW, fcb):
    B, T = input_seq.shape[0], input_seq.shape[1]
    BS = B // _GRID
    xseq = (input_seq.transpose(1, 2, 0, 3)
            .reshape(T, _N, _GRID, BS * _D_IN)
            .transpose(0, 2, 1, 3))
    idx = jnp.clip(seq_lengths - 1, 0, T - 1).astype(jnp.int32)
    oh = (idx[None, :] == jnp.arange(T, dtype=jnp.int32)[:, None])
    oh = oh.astype(jnp.float32).reshape(T, 1, B, 1)

    c0 = _D_IN + _HID
    c1 = 2 * _HID
    f32 = jnp.float32

    def fixed(shape):
        nd = len(shape)
        return pl.BlockSpec(shape, lambda i, _n=nd: (0,) * _n)

    bf16 = jnp.bfloat16
    w1g0x, w1g0h = _prep_w(W1g0.astype(bf16), c0)
    w1c0x, w1c0h = _prep_w(W1c0.astype(bf16), c0)
    w1g1x, w1g1h = _prep_w(W1g1.astype(bf16), c1)
    w1c1x, w1c1h = _prep_w(W1c1.astype(bf16), c1)
    w2g0x, w2g0h = _prep_w(W2g0.astype(bf16), c0)
    w2c0x, w2c0h = _prep_w(W2c0.astype(bf16), c0)
    w2g1x, w2g1h = _prep_w(W2g1.astype(bf16), c1)
    w2c1x, w2c1h = _prep_w(W2c1.astype(bf16), c1)

    args = (
        xseq, oh, supports1.astype(bf16), supports2.astype(bf16),
        w1g0x, w1g0h, b1g0.reshape(1, 1, -1),
        w1c0x, w1c0h, b1c0.reshape(1, 1, -1),
        w1g1x, w1g1h, b1g1.reshape(1, 1, -1),
        w1c1x, w1c1h, b1c1.reshape(1, 1, -1),
        w2g0x, w2g0h, b2g0.reshape(1, 1, -1),
        w2c0x, w2c0h, b2c0.reshape(1, 1, -1),
        w2g1x, w2g1h, b2g1.reshape(1, 1, -1),
        w2c1x, w2c1h, b2c1.reshape(1, 1, -1),
        jnp.concatenate([fcW[:_HID], fcW[_HID:]],
                        axis=1).astype(bf16),
        fcb.reshape(1, -1),
    )
    in_specs = [
        pl.BlockSpec((T, 1, _N, BS * _D_IN), lambda i: (0, i, 0, 0)),
        pl.BlockSpec((T, 1, BS, 1), lambda i: (0, 0, i, 0)),
    ] + [fixed(a.shape) for a in args[2:]]

    out = pl.pallas_call(
        _body,
        grid=(_GRID,),
        out_shape=jax.ShapeDtypeStruct((B, _CLS), f32),
        in_specs=in_specs,
        out_specs=pl.BlockSpec((BS, _CLS), lambda i: (i, 0)),
        scratch_shapes=[
            pltpu.VMEM((_N, BS, 3 * _HID), f32),
            pltpu.VMEM((_N, BS, _HID), f32),
            pltpu.VMEM((_N, BS, _HID), f32),
            pltpu.VMEM((_N, BS, _HID), f32),
            pltpu.VMEM((_N, BS, 3 * _HID), f32),
            pltpu.VMEM((_N, BS, _HID), f32),
            pltpu.VMEM((_N, BS, _HID), f32),
            pltpu.VMEM((_N, BS, _HID), f32),
        ],
        compiler_params=pltpu.CompilerParams(
            vmem_limit_bytes=100 * 1024 * 1024),
    )(*args)
    return out


# speedup vs baseline: 9.0474x; 1.1325x over previous
import jax
import jax.numpy as jnp
from jax.experimental import pallas as pl
from jax.experimental.pallas import tpu as pltpu

_N = 207
_D_IN = 2
_HID = 64
_CLS = 4
_GRID = 2


def _sdot(S, x):
    return jax.lax.dot_general(S, x, (((1,), (0,)), ((), ())),
                               preferred_element_type=jnp.float32)


def _cdot(x, w):
    return jax.lax.dot_general(x, w, (((2,), (0,)), ((), ())),
                               preferred_element_type=jnp.float32)


def _body(x_ref, oh_ref, s1_ref, s2_ref,
          w1g0x_ref, w1g0h_ref, b1g0_ref, w1c0x_ref, w1c0h_ref, b1c0_ref,
          w1g1_ref, b1g1_ref, w1c1x_ref, w1c1h_ref, b1c1_ref,
          w2g0x_ref, w2g0h_ref, b2g0_ref, w2c0x_ref, w2c0h_ref, b2c0_ref,
          w2g1_ref, b2g1_ref, w2c1x_ref, w2c1h_ref, b2c1_ref,
          fcw_ref, fcb_ref, o_ref,
          ch1_ref, h01_ref, h11_ref, l1_ref,
          ch2_ref, h02_ref, h12_ref, l2_ref):
    T = x_ref.shape[0]
    B = oh_ref.shape[2]
    bf = jnp.bfloat16

    def chebcat(S, x0f):
        x0 = x0f.astype(bf)
        x1f = _sdot(S, x0)
        x1 = x1f.astype(bf)
        x2 = (2.0 * _sdot(S, x1) - x0f).astype(bf)
        return jnp.concatenate([x0, x1, x2], axis=-1)

    for ref in (ch1_ref, ch2_ref, h01_ref, h02_ref,
                h11_ref, h12_ref, l1_ref, l2_ref):
        ref[...] = jnp.zeros_like(ref)

    encs = (
        (s1_ref, ch1_ref, h01_ref, h11_ref, l1_ref,
         w1g0x_ref, w1g0h_ref, b1g0_ref, w1c0x_ref, w1c0h_ref, b1c0_ref,
         w1g1_ref, b1g1_ref, w1c1x_ref, w1c1h_ref, b1c1_ref),
        (s2_ref, ch2_ref, h02_ref, h12_ref, l2_ref,
         w2g0x_ref, w2g0h_ref, b2g0_ref, w2c0x_ref, w2c0h_ref, b2c0_ref,
         w2g1_ref, b2g1_ref, w2c1x_ref, w2c1h_ref, b2c1_ref),
    )

    def step(t, carry):
        xw_t = x_ref[t, 0]
        x_tf = xw_t.reshape(_N, B, _D_IN)
        m = oh_ref[t]
        for (s_ref, ch_ref, h0_ref, h1_ref, l_ref,
             wg0x, wg0h, bg0, wc0x, wc0h, bc0,
             wg1, bg1, wc1x, wc1h, bc1) in encs:
            S = s_ref[...]
            xcat0 = chebcat(S, x_tf)
            cc_prev = ch_ref[...]
            h0 = h0_ref[...]
            g = jax.nn.sigmoid(_cdot(xcat0, wg0x[...]) +
                               _cdot(cc_prev, wg0h[...]) + bg0[...])
            r = g[:, :, :_HID]
            u = g[:, :, _HID:]
            ccat = chebcat(S, r * h0)
            c = jnp.tanh(_cdot(xcat0, wc0x[...]) +
                         _cdot(ccat, wc0h[...]) + bc0[...])
            h0n = u * h0 + (1.0 - u) * c
            h1 = h1_ref[...]
            cc = chebcat(S, jnp.concatenate([h0n, h1], axis=-1))
            ch_ref[...] = cc
            h0_ref[...] = h0n
            g1 = jax.nn.sigmoid(_cdot(cc, wg1[...]) + bg1[...])
            r1 = g1[:, :, :_HID]
            u1 = g1[:, :, _HID:]
            ccat1 = chebcat(S, r1 * h1)
            c1 = jnp.tanh(_cdot(cc, wc1x[...]) +
                          _cdot(ccat1, wc1h[...]) + bc1[...])
            h1n = u1 * h1 + (1.0 - u1) * c1
            h1_ref[...] = h1n
            l_ref[...] = m * h1n + (1.0 - m) * l_ref[...]
        return carry

    jax.lax.fori_loop(0, T, step, 0, unroll=False)

    ua = jnp.maximum(
        jnp.concatenate([l1_ref[...], l2_ref[...]], axis=0),
        0.0).astype(bf)
    pq = _cdot(ua, fcw_ref[...])
    pq = pq.reshape(_N, 2, B, 2 * _CLS)
    logits = pq[:, 0, :, :_CLS] + pq[:, 1, :, _CLS:]
    o_ref[...] = jnp.max(logits, axis=0) + fcb_ref[...]


def _term_major(w3):
    return w3.transpose(1, 0, 2).reshape(3 * w3.shape[0], w3.shape[2])


def kernel(input_seq, seq_lengths, supports1, supports2,
           W1g0, b1g0, W1c0, b1c0, W1g1, b1g1, W1c1, b1c1,
           W2g0, b2g0, W2c0, b2c0, W2g1, b2g1, W2c1, b2c1,
           fcW, fcb):
    B, T = input_seq.shape[0], input_seq.shape[1]
    BS = B // _GRID
    xseq = (input_seq.transpose(1, 2, 0, 3)
            .reshape(T, _N, _GRID, BS * _D_IN)
            .transpose(0, 2, 1, 3))
    idx = jnp.clip(seq_lengths - 1, 0, T - 1).astype(jnp.int32)
    oh = (idx[None, :] == jnp.arange(T, dtype=jnp.int32)[:, None])
    oh = oh.astype(jnp.float32).reshape(T, 1, B, 1)

    c0 = _D_IN + _HID
    c1 = 2 * _HID
    f32 = jnp.float32
    bf16 = jnp.bfloat16

    def split_xh(W, cin):
        w3 = W.reshape(cin, 3, W.shape[1]).astype(bf16)
        return w3[:cin - _HID], w3[cin - _HID:]

    def pad_interleave(wa3):
        w6 = jnp.concatenate([wa3, jnp.zeros_like(wa3)], axis=0)
        return _term_major(w6)

    def prep_layer0(Wg, Wc):
        gx3, gh3 = split_xh(Wg, c0)
        cx3, ch3 = split_xh(Wc, c0)
        return (_term_major(gx3), pad_interleave(gh3),
                _term_major(cx3), _term_major(ch3))

    def prep_layer1(Wg, Wc):
        g3 = Wg.reshape(c1, 3, Wg.shape[1]).astype(bf16)
        cx3, ch3 = split_xh(Wc, c1)
        return (_term_major(g3), pad_interleave(cx3),
                _term_major(ch3))

    w1g0x, w1g0h, w1c0x, w1c0h = prep_layer0(W1g0, W1c0)
    w1g1, w1c1x, w1c1h = prep_layer1(W1g1, W1c1)
    w2g0x, w2g0h, w2c0x, w2c0h = prep_layer0(W2g0, W2c0)
    w2g1, w2c1x, w2c1h = prep_layer1(W2g1, W2c1)

    args = (
        xseq, oh, supports1.astype(bf16), supports2.astype(bf16),
        w1g0x, w1g0h, b1g0.reshape(1, 1, -1),
        w1c0x, w1c0h, b1c0.reshape(1, 1, -1),
        w1g1, b1g1.reshape(1, 1, -1),
        w1c1x, w1c1h, b1c1.reshape(1, 1, -1),
        w2g0x, w2g0h, b2g0.reshape(1, 1, -1),
        w2c0x, w2c0h, b2c0.reshape(1, 1, -1),
        w2g1, b2g1.reshape(1, 1, -1),
        w2c1x, w2c1h, b2c1.reshape(1, 1, -1),
        jnp.concatenate([fcW[:_HID], fcW[_HID:]],
                        axis=1).astype(bf16),
        fcb.reshape(1, -1),
    )
    in_specs = [
        pl.BlockSpec((T, 1, _N, BS * _D_IN), lambda i: (0, i, 0, 0)),
        pl.BlockSpec((T, 1, BS, 1), lambda i: (0, 0, i, 0)),
    ] + [pl.BlockSpec(a.shape, lambda i, _n=len(a.shape): (0,) * _n)
         for a in args[2:]]

    out = pl.pallas_call(
        _body,
        grid=(_GRID,),
        out_shape=jax.ShapeDtypeStruct((B, _CLS), f32),
        in_specs=in_specs,
        out_specs=pl.BlockSpec((BS, _CLS), lambda i: (i, 0)),
        scratch_shapes=[
            pltpu.VMEM((_N, BS, 6 * _HID), bf16),
            pltpu.VMEM((_N, BS, _HID), f32),
            pltpu.VMEM((_N, BS, _HID), f32),
            pltpu.VMEM((_N, BS, _HID), f32),
            pltpu.VMEM((_N, BS, 6 * _HID), bf16),
            pltpu.VMEM((_N, BS, _HID), f32),
            pltpu.VMEM((_N, BS, _HID), f32),
            pltpu.VMEM((_N, BS, _HID), f32),
        ],
        compiler_params=pltpu.CompilerParams(
            vmem_limit_bytes=100 * 1024 * 1024),
    )(*args)
    return out


# speedup vs baseline: 9.5048x; 1.0505x over previous
import jax
import jax.numpy as jnp
from jax.experimental import pallas as pl
from jax.experimental.pallas import tpu as pltpu

_N = 207
_D_IN = 2
_HID = 64
_CLS = 4
_GRID = 2


def _sdot(S, x):
    return jax.lax.dot_general(S, x, (((1,), (0,)), ((), ())),
                               preferred_element_type=jnp.float32)


def _cdot(x, w):
    return jax.lax.dot_general(x, w, (((2,), (0,)), ((), ())),
                               preferred_element_type=jnp.float32)


def _body(x_ref, oh_ref, s1_ref, s2_ref,
          w1gc0x_ref, w1g0h_ref, b1g0_ref, w1c0h_ref, b1c0_ref,
          w1gc1_ref, b1g1_ref, w1c1h_ref, b1c1_ref,
          w2gc0x_ref, w2g0h_ref, b2g0_ref, w2c0h_ref, b2c0_ref,
          w2gc1_ref, b2g1_ref, w2c1h_ref, b2c1_ref,
          fcw_ref, fcb_ref, o_ref,
          ch1_ref, h01_ref, h11_ref, l1_ref,
          ch2_ref, h02_ref, h12_ref, l2_ref):
    T = x_ref.shape[0]
    B = oh_ref.shape[2]
    bf = jnp.bfloat16

    def chebcat(S, x0f):
        x0 = x0f.astype(bf)
        x1f = _sdot(S, x0)
        x1 = x1f.astype(bf)
        x2 = (2.0 * _sdot(S, x1) - x0f).astype(bf)
        return jnp.concatenate([x0, x1, x2], axis=-1)

    for ref in (ch1_ref, ch2_ref, h01_ref, h02_ref,
                h11_ref, h12_ref, l1_ref, l2_ref):
        ref[...] = jnp.zeros_like(ref)

    encs = (
        (s1_ref, ch1_ref, h01_ref, h11_ref, l1_ref,
         w1gc0x_ref, w1g0h_ref, b1g0_ref, w1c0h_ref, b1c0_ref,
         w1gc1_ref, b1g1_ref, w1c1h_ref, b1c1_ref),
        (s2_ref, ch2_ref, h02_ref, h12_ref, l2_ref,
         w2gc0x_ref, w2g0h_ref, b2g0_ref, w2c0h_ref, b2c0_ref,
         w2gc1_ref, b2g1_ref, w2c1h_ref, b2c1_ref),
    )

    def step(t, carry):
        xw_t = x_ref[t, 0]
        x_tf = xw_t.reshape(_N, B, _D_IN)
        m = oh_ref[t]
        for (s_ref, ch_ref, h0_ref, h1_ref, l_ref,
             wgc0x, wg0h, bg0, wc0h, bc0,
             wgc1, bg1, wc1h, bc1) in encs:
            S = s_ref[...]
            xcat0 = chebcat(S, x_tf)
            cc_prev = ch_ref[...]
            h0 = h0_ref[...]
            px = _cdot(xcat0, wgc0x[...])
            g = jax.nn.sigmoid(px[:, :, :2 * _HID] +
                               _cdot(cc_prev, wg0h[...]) + bg0[...])
            r = g[:, :, :_HID]
            u = g[:, :, _HID:]
            ccat = chebcat(S, r * h0)
            c = jnp.tanh(px[:, :, 2 * _HID:] +
                         _cdot(ccat, wc0h[...]) + bc0[...])
            h0n = c + u * (h0 - c)
            h1 = h1_ref[...]
            cc = chebcat(S, jnp.concatenate([h0n, h1], axis=-1))
            ch_ref[...] = cc
            h0_ref[...] = h0n
            p1 = _cdot(cc, wgc1[...])
            g1 = jax.nn.sigmoid(p1[:, :, :2 * _HID] + bg1[...])
            r1 = g1[:, :, :_HID]
            u1 = g1[:, :, _HID:]
            ccat1 = chebcat(S, r1 * h1)
            c1 = jnp.tanh(p1[:, :, 2 * _HID:] +
                          _cdot(ccat1, wc1h[...]) + bc1[...])
            h1n = c1 + u1 * (h1 - c1)
            h1_ref[...] = h1n
            l_ref[...] = m * h1n + (1.0 - m) * l_ref[...]
        return carry

    jax.lax.fori_loop(0, T, step, 0, unroll=False)

    ua = jnp.maximum(
        jnp.concatenate([l1_ref[...], l2_ref[...]], axis=0),
        0.0).astype(bf)
    pq = _cdot(ua, fcw_ref[...])
    pq = pq.reshape(_N, 2, B, 2 * _CLS)
    logits = pq[:, 0, :, :_CLS] + pq[:, 1, :, _CLS:]
    o_ref[...] = jnp.max(logits, axis=0) + fcb_ref[...]


def _term_major(w3):
    return w3.transpose(1, 0, 2).reshape(3 * w3.shape[0], w3.shape[2])


def kernel(input_seq, seq_lengths, supports1, supports2,
           W1g0, b1g0, W1c0, b1c0, W1g1, b1g1, W1c1, b1c1,
           W2g0, b2g0, W2c0, b2c0, W2g1, b2g1, W2c1, b2c1,
           fcW, fcb):
    B, T = input_seq.shape[0], input_seq.shape[1]
    BS = B // _GRID
    xseq = (input_seq.transpose(1, 2, 0, 3)
            .reshape(T, _N, _GRID, BS * _D_IN)
            .transpose(0, 2, 1, 3))
    idx = jnp.clip(seq_lengths - 1, 0, T - 1).astype(jnp.int32)
    oh = (idx[None, :] == jnp.arange(T, dtype=jnp.int32)[:, None])
    oh = oh.astype(jnp.float32).reshape(T, 1, B, 1)

    c0 = _D_IN + _HID
    c1 = 2 * _HID
    f32 = jnp.float32
    bf16 = jnp.bfloat16

    def split_xh(W, cin):
        w3 = W.reshape(cin, 3, W.shape[1]).astype(bf16)
        return w3[:cin - _HID], w3[cin - _HID:]

    def pad_interleave(wa3):
        w6 = jnp.concatenate([wa3, jnp.zeros_like(wa3)], axis=0)
        return _term_major(w6)

    def prep_layer0(Wg, Wc):
        gx3, gh3 = split_xh(Wg, c0)
        cx3, ch3 = split_xh(Wc, c0)
        gcx = jnp.concatenate([_term_major(gx3), _term_major(cx3)], axis=1)
        return gcx, pad_interleave(gh3), _term_major(ch3)

    def prep_layer1(Wg, Wc):
        g3 = Wg.reshape(c1, 3, Wg.shape[1]).astype(bf16)
        cx3, ch3 = split_xh(Wc, c1)
        gc = jnp.concatenate([_term_major(g3), pad_interleave(cx3)], axis=1)
        return gc, _term_major(ch3)

    w1gc0x, w1g0h, w1c0h = prep_layer0(W1g0, W1c0)
    w1gc1, w1c1h = prep_layer1(W1g1, W1c1)
    w2gc0x, w2g0h, w2c0h = prep_layer0(W2g0, W2c0)
    w2gc1, w2c1h = prep_layer1(W2g1, W2c1)

    args = (
        xseq, oh, supports1.astype(bf16), supports2.astype(bf16),
        w1gc0x, w1g0h, b1g0.reshape(1, 1, -1),
        w1c0h, b1c0.reshape(1, 1, -1),
        w1gc1, b1g1.reshape(1, 1, -1),
        w1c1h, b1c1.reshape(1, 1, -1),
        w2gc0x, w2g0h, b2g0.reshape(1, 1, -1),
        w2c0h, b2c0.reshape(1, 1, -1),
        w2gc1, b2g1.reshape(1, 1, -1),
        w2c1h, b2c1.reshape(1, 1, -1),
        jnp.concatenate([fcW[:_HID], fcW[_HID:]],
                        axis=1).astype(bf16),
        fcb.reshape(1, -1),
    )
    in_specs = [
        pl.BlockSpec((T, 1, _N, BS * _D_IN), lambda i: (0, i, 0, 0)),
        pl.BlockSpec((T, 1, BS, 1), lambda i: (0, 0, i, 0)),
    ] + [pl.BlockSpec(a.shape, lambda i, _n=len(a.shape): (0,) * _n)
         for a in args[2:]]

    out = pl.pallas_call(
        _body,
        grid=(_GRID,),
        out_shape=jax.ShapeDtypeStruct((B, _CLS), f32),
        in_specs=in_specs,
        out_specs=pl.BlockSpec((BS, _CLS), lambda i: (i, 0)),
        scratch_shapes=[
            pltpu.VMEM((_N, BS, 6 * _HID), bf16),
            pltpu.VMEM((_N, BS, _HID), f32),
            pltpu.VMEM((_N, BS, _HID), f32),
            pltpu.VMEM((_N, BS, _HID), f32),
            pltpu.VMEM((_N, BS, 6 * _HID), bf16),
            pltpu.VMEM((_N, BS, _HID), f32),
            pltpu.VMEM((_N, BS, _HID), f32),
            pltpu.VMEM((_N, BS, _HID), f32),
        ],
        compiler_params=pltpu.CompilerParams(
            vmem_limit_bytes=100 * 1024 * 1024),
    )(*args)
    return out


# speedup vs baseline: 9.5175x; 1.0013x over previous
import jax
import jax.numpy as jnp
from jax.experimental import pallas as pl
from jax.experimental.pallas import tpu as pltpu

_N = 207
_D_IN = 2
_HID = 64
_CLS = 4
_GRID = 2


def _sdot(S, x):
    return jax.lax.dot_general(S, x, (((1,), (0,)), ((), ())),
                               preferred_element_type=jnp.float32)


def _cdot(x, w):
    return jax.lax.dot_general(x, w, (((2,), (0,)), ((), ())),
                               preferred_element_type=jnp.float32)


def _body(x_ref, oh_ref, s1_ref, s2_ref,
          w1gc0x_ref, w1g0h_ref, b1g0_ref, w1c0h_ref, b1c0_ref,
          w1gc1_ref, b1g1_ref, w1c1h_ref, b1c1_ref,
          w2gc0x_ref, w2g0h_ref, b2g0_ref, w2c0h_ref, b2c0_ref,
          w2gc1_ref, b2g1_ref, w2c1h_ref, b2c1_ref,
          fcw_ref, fcb_ref, o_ref,
          ch1_ref, h01_ref, h11_ref, l1_ref,
          ch2_ref, h02_ref, h12_ref, l2_ref):
    T = x_ref.shape[0]
    B = oh_ref.shape[2]
    bf = jnp.bfloat16

    def chebcat(S, x0f):
        x0 = x0f.astype(bf)
        x1f = _sdot(S, x0)
        x1 = x1f.astype(bf)
        x2 = (2.0 * _sdot(S, x1) - x0f).astype(bf)
        return jnp.concatenate([x0, x1, x2], axis=-1)

    for ref in (ch1_ref, ch2_ref, h01_ref, h02_ref,
                h11_ref, h12_ref, l1_ref, l2_ref):
        ref[...] = jnp.zeros_like(ref)

    encs = (
        (s1_ref, ch1_ref, h01_ref, h11_ref, l1_ref,
         w1gc0x_ref, w1g0h_ref, b1g0_ref, w1c0h_ref, b1c0_ref,
         w1gc1_ref, b1g1_ref, w1c1h_ref, b1c1_ref),
        (s2_ref, ch2_ref, h02_ref, h12_ref, l2_ref,
         w2gc0x_ref, w2g0h_ref, b2g0_ref, w2c0h_ref, b2c0_ref,
         w2gc1_ref, b2g1_ref, w2c1h_ref, b2c1_ref),
    )

    def step(t, carry):
        xw_t = x_ref[t, 0]
        x_tf = xw_t.reshape(_N, B, _D_IN)
        m = oh_ref[t]
        for (s_ref, ch_ref, h0_ref, h1_ref, l_ref,
             wgc0x, wg0h, bg0, wc0h, bc0,
             wgc1, bg1, wc1h, bc1) in encs:
            S = s_ref[...]
            xcat0 = chebcat(S, x_tf)
            cc_prev = ch_ref[...]
            h0 = h0_ref[...]
            px = _cdot(xcat0, wgc0x[...])
            g = jax.nn.sigmoid(px[:, :, :2 * _HID] +
                               _cdot(cc_prev, wg0h[...]) + bg0[...])
            r = g[:, :, :_HID]
            u = g[:, :, _HID:]
            ccat = chebcat(S, r * h0)
            c = jnp.tanh(px[:, :, 2 * _HID:] +
                         _cdot(ccat, wc0h[...]) + bc0[...])
            h0n = c + u * (h0 - c)
            h1 = h1_ref[...]
            cc = chebcat(S, jnp.concatenate([h0n, h1], axis=-1))
            ch_ref[...] = cc
            h0_ref[...] = h0n
            p1 = _cdot(cc, wgc1[...])
            g1 = jax.nn.sigmoid(p1[:, :, :2 * _HID] + bg1[...])
            r1 = g1[:, :, :_HID]
            u1 = g1[:, :, _HID:]
            ccat1 = chebcat(S, r1 * h1)
            c1 = jnp.tanh(p1[:, :, 2 * _HID:] +
                          _cdot(ccat1, wc1h[...]) + bc1[...])
            h1n = c1 + u1 * (h1 - c1)
            h1_ref[...] = h1n
            lv = l_ref[...]
            l_ref[...] = lv + m * (h1n - lv)
        return carry

    jax.lax.fori_loop(0, T, step, 0, unroll=False)

    ua = jnp.maximum(
        jnp.concatenate([l1_ref[...], l2_ref[...]], axis=0),
        0.0).astype(bf)
    pq = _cdot(ua, fcw_ref[...])
    pq = pq.reshape(_N, 2, B, 2 * _CLS)
    logits = pq[:, 0, :, :_CLS] + pq[:, 1, :, _CLS:]
    o_ref[...] = jnp.max(logits, axis=0) + fcb_ref[...]


def _term_major(w3):
    return w3.transpose(1, 0, 2).reshape(3 * w3.shape[0], w3.shape[2])


def kernel(input_seq, seq_lengths, supports1, supports2,
           W1g0, b1g0, W1c0, b1c0, W1g1, b1g1, W1c1, b1c1,
           W2g0, b2g0, W2c0, b2c0, W2g1, b2g1, W2c1, b2c1,
           fcW, fcb):
    B, T = input_seq.shape[0], input_seq.shape[1]
    BS = B // _GRID
    xseq = (input_seq.transpose(1, 2, 0, 3)
            .reshape(T, _N, _GRID, BS * _D_IN)
            .transpose(0, 2, 1, 3))
    idx = jnp.clip(seq_lengths - 1, 0, T - 1).astype(jnp.int32)
    oh = (idx[None, :] == jnp.arange(T, dtype=jnp.int32)[:, None])
    oh = oh.astype(jnp.float32).reshape(T, 1, B, 1)

    c0 = _D_IN + _HID
    c1 = 2 * _HID
    f32 = jnp.float32
    bf16 = jnp.bfloat16

    def split_xh(W, cin):
        w3 = W.reshape(cin, 3, W.shape[1]).astype(bf16)
        return w3[:cin - _HID], w3[cin - _HID:]

    def pad_interleave(wa3):
        w6 = jnp.concatenate([wa3, jnp.zeros_like(wa3)], axis=0)
        return _term_major(w6)

    def prep_layer0(Wg, Wc):
        gx3, gh3 = split_xh(Wg, c0)
        cx3, ch3 = split_xh(Wc, c0)
        gcx = jnp.concatenate([_term_major(gx3), _term_major(cx3)], axis=1)
        return gcx, pad_interleave(gh3), _term_major(ch3)

    def prep_layer1(Wg, Wc):
        g3 = Wg.reshape(c1, 3, Wg.shape[1]).astype(bf16)
        cx3, ch3 = split_xh(Wc, c1)
        gc = jnp.concatenate([_term_major(g3), pad_interleave(cx3)], axis=1)
        return gc, _term_major(ch3)

    w1gc0x, w1g0h, w1c0h = prep_layer0(W1g0, W1c0)
    w1gc1, w1c1h = prep_layer1(W1g1, W1c1)
    w2gc0x, w2g0h, w2c0h = prep_layer0(W2g0, W2c0)
    w2gc1, w2c1h = prep_layer1(W2g1, W2c1)

    args = (
        xseq, oh, supports1.astype(bf16), supports2.astype(bf16),
        w1gc0x, w1g0h, b1g0.reshape(1, 1, -1),
        w1c0h, b1c0.reshape(1, 1, -1),
        w1gc1, b1g1.reshape(1, 1, -1),
        w1c1h, b1c1.reshape(1, 1, -1),
        w2gc0x, w2g0h, b2g0.reshape(1, 1, -1),
        w2c0h, b2c0.reshape(1, 1, -1),
        w2gc1, b2g1.reshape(1, 1, -1),
        w2c1h, b2c1.reshape(1, 1, -1),
        jnp.concatenate([fcW[:_HID], fcW[_HID:]],
                        axis=1).astype(bf16),
        fcb.reshape(1, -1),
    )
    in_specs = [
        pl.BlockSpec((T, 1, _N, BS * _D_IN), lambda i: (0, i, 0, 0)),
        pl.BlockSpec((T, 1, BS, 1), lambda i: (0, 0, i, 0)),
    ] + [pl.BlockSpec(a.shape, lambda i, _n=len(a.shape): (0,) * _n)
         for a in args[2:]]

    out = pl.pallas_call(
        _body,
        grid=(_GRID,),
        out_shape=jax.ShapeDtypeStruct((B, _CLS), f32),
        in_specs=in_specs,
        out_specs=pl.BlockSpec((BS, _CLS), lambda i: (i, 0)),
        scratch_shapes=[
            pltpu.VMEM((_N, BS, 6 * _HID), bf16),
            pltpu.VMEM((_N, BS, _HID), f32),
            pltpu.VMEM((_N, BS, _HID), f32),
            pltpu.VMEM((_N, BS, _HID), f32),
            pltpu.VMEM((_N, BS, 6 * _HID), bf16),
            pltpu.VMEM((_N, BS, _HID), f32),
            pltpu.VMEM((_N, BS, _HID), f32),
            pltpu.VMEM((_N, BS, _HID), f32),
        ],
        compiler_params=pltpu.CompilerParams(
            vmem_limit_bytes=100 * 1024 * 1024),
    )(*args)
    return out
